# Initial kernel scaffold; baseline (speedup 1.0000x reference)
import jax
import jax.numpy as jnp
from jax.experimental import pallas as pl


def kernel(x, edge_index, W_gc1, b_gc1, W_res1, b_res1, gamma1, beta1, W_gc2, b_gc2, W_res2, b_res2, gamma2, beta2, W_aw, b_aw, W_p1, b_p1, gamma_p, beta_p, W_p2, b_p2):
    raise NotImplementedError("write your pallas kernel here")


# speedup vs baseline: 5.8174x; 5.8174x over previous
import functools

import jax
import jax.numpy as jnp
from jax import lax
from jax.experimental import pallas as pl
from jax.experimental.pallas import tpu as pltpu
from jax.experimental.pallas import tpu_sc as plsc

N = 10000
E = 320000
D_IN = 128
H = 64
PH = 128

NC = 2
NS = 16
NW = NC * NS

CH = 128
NCH = -(-E // (NW * CH))
E_PAD = NW * NCH * CH
NP = ((N + NS * 8 - 1) // (NS * 8)) * (NS * 8)
RPT = NP // NS


def _tc_pre(x_ref, wg_ref, wr_ref, br_ref, xw_ref, r_ref):
    x = x_ref[...]
    xw_ref[...] = jnp.dot(x, wg_ref[...], preferred_element_type=jnp.float32)
    r_ref[...] = jnp.maximum(
        jnp.dot(x, wr_ref[...], preferred_element_type=jnp.float32) + br_ref[...], 0.0)


def _tc_mid(aggp_ref, r1_ref, bg_ref, g_ref, be_ref, wg2_ref, wr2_ref, br2_ref,
            xw2_ref, r2_ref):
    agg = aggp_ref[0, :N, :] + aggp_ref[1, :N, :]
    h1 = g_ref[...] * (jnp.maximum(agg + bg_ref[...], 0.0) + r1_ref[...]) + be_ref[...]
    xw2_ref[...] = jnp.dot(h1, wg2_ref[...], preferred_element_type=jnp.float32)
    r2_ref[...] = jnp.maximum(
        jnp.dot(h1, wr2_ref[...], preferred_element_type=jnp.float32) + br2_ref[...], 0.0)


def _tc_post(aggp_ref, r2_ref, bg_ref, g_ref, be_ref, waw_ref, baw_ref,
             wp1_ref, bp1_ref, gp_ref, bep_ref, wp2_ref, bp2_ref,
             pred_ref, gf_ref):
    agg = aggp_ref[0, :N, :] + aggp_ref[1, :N, :]
    h = g_ref[...] * (jnp.maximum(agg + bg_ref[...], 0.0) + r2_ref[...]) + be_ref[...]
    w = jax.nn.sigmoid(jnp.sum(h * waw_ref[...], axis=1, keepdims=True) + baw_ref[0, 0])
    hsum = jnp.sum(h * w, axis=0, keepdims=True)
    hmax = jnp.max(h, axis=0, keepdims=True)
    gf = jnp.concatenate([hsum, hmax], axis=1)
    gf_ref[...] = gf
    z = jnp.maximum(
        jnp.dot(gf, wp1_ref[...], preferred_element_type=jnp.float32) + bp1_ref[...], 0.0)
    z = gp_ref[...] * z + bep_ref[...]
    pred_ref[...] = jnp.sum(z * wp2_ref[...], axis=1, keepdims=True) + bp2_ref[0, 0]


@functools.lru_cache(maxsize=None)
def _make_sc_agg():
    @functools.partial(
        pl.kernel,
        out_type=jax.ShapeDtypeStruct((NC, NP, H), jnp.float32),
        mesh=plsc.VectorSubcoreMesh(core_axis_name="c", subcore_axis_name="s",
                                    num_cores=NC, num_subcores=NS),
        scratch_types=[
            pltpu.VMEM((NCH, CH), jnp.int32),
            pltpu.VMEM((NCH, CH), jnp.int32),
            pltpu.VMEM((CH, H), jnp.float32),
            pltpu.VMEM_SHARED((NP, H), jnp.float32),
            pltpu.SemaphoreType.DMA,
        ],
        compiler_params=pltpu.CompilerParams(use_tc_tiling_on_sc=False),
    )
    def _sc_agg(xw_hbm, src_hbm, dst_hbm, zeros_hbm, out_hbm,
                src_v, dst_v, rows_v, acc_sh, sem):
        c = lax.axis_index("c")
        s = lax.axis_index("s")
        wid = s * NC + c
        pltpu.sync_copy(zeros_hbm.at[pl.ds(s * RPT, RPT)],
                        acc_sh.at[pl.ds(s * RPT, RPT)])
        pltpu.sync_copy(src_hbm.at[wid], src_v)
        pltpu.sync_copy(dst_hbm.at[wid], dst_v)
        plsc.subcore_barrier()

        def body(j, carry):
            pltpu.async_copy(xw_hbm.at[src_v.at[j]], rows_v, sem).wait()
            pltpu.sync_copy(rows_v, acc_sh.at[dst_v.at[j]], add=True)
            return carry

        lax.fori_loop(0, NCH, body, 0)
        plsc.subcore_barrier()
        pltpu.sync_copy(acc_sh.at[pl.ds(s * RPT, RPT)],
                        out_hbm.at[c, pl.ds(s * RPT, RPT)])

    return _sc_agg


def kernel(x, edge_index, W_gc1, b_gc1, W_res1, b_res1, gamma1, beta1,
           W_gc2, b_gc2, W_res2, b_res2, gamma2, beta2, W_aw, b_aw,
           W_p1, b_p1, gamma_p, beta_p, W_p2, b_p2):
    src = edge_index[0]
    dst = edge_index[1]
    pad = E_PAD - E
    src3 = jnp.concatenate([src, jnp.zeros((pad,), jnp.int32)]).reshape(NW, NCH, CH)
    dst3 = jnp.concatenate([dst, jnp.full((pad,), N, jnp.int32)]).reshape(NW, NCH, CH)
    zeros_np = jnp.zeros((NP, H), jnp.float32)

    xw1, r1 = pl.pallas_call(
        _tc_pre,
        out_shape=[jax.ShapeDtypeStruct((N, H), jnp.float32),
                   jax.ShapeDtypeStruct((N, H), jnp.float32)],
    )(x, W_gc1, W_res1, b_res1.reshape(1, H))

    aggp1 = _make_sc_agg()(xw1, src3, dst3, zeros_np)

    xw2, r2 = pl.pallas_call(
        _tc_mid,
        out_shape=[jax.ShapeDtypeStruct((N, H), jnp.float32),
                   jax.ShapeDtypeStruct((N, H), jnp.float32)],
    )(aggp1, r1, b_gc1.reshape(1, H), gamma1.reshape(1, H), beta1.reshape(1, H),
      W_gc2, W_res2, b_res2.reshape(1, H))

    aggp2 = _make_sc_agg()(xw2, src3, dst3, zeros_np)

    pred, gf = pl.pallas_call(
        _tc_post,
        out_shape=[jax.ShapeDtypeStruct((1, 1), jnp.float32),
                   jax.ShapeDtypeStruct((1, 2 * H), jnp.float32)],
    )(aggp2, r2, b_gc2.reshape(1, H), gamma2.reshape(1, H), beta2.reshape(1, H),
      W_aw.reshape(1, H), b_aw.reshape(1, 1),
      W_p1, b_p1.reshape(1, PH), gamma_p.reshape(1, PH), beta_p.reshape(1, PH),
      W_p2.reshape(1, PH), b_p2.reshape(1, 1))

    return (pred, gf)
